# baseline (speedup 1.0000x reference)
import functools

import jax
import jax.numpy as jnp
from jax import lax
from jax.experimental import pallas as pl
from jax.experimental.pallas import tpu as pltpu
from jax.experimental.pallas import tpu_sc as plsc

_LANES = 16
_TAB = 32


def _pad128(n):
    return (n + 127) // 128 * 128


@functools.partial(
    jax.jit,
    static_argnames=("rows_per_chunk", "n_channels", "nw", "n_cores", "np1"),
)
def _pwlu_sc(x, ab_tab, lo_tab, inv_tab, *, rows_per_chunk,
             n_channels, nw, n_cores, np1):
    bsz, _, h, w = x.shape
    chunks_per_plane = h // rows_per_chunk
    n_chunks = bsz * n_channels * chunks_per_plane
    chunks_per_tile = n_chunks // nw
    n_pairs = chunks_per_tile // 2
    mesh = plsc.VectorSubcoreMesh(core_axis_name="c", subcore_axis_name="s")

    @functools.partial(
        pl.kernel,
        out_type=jax.ShapeDtypeStruct(x.shape, jnp.float32),
        mesh=mesh,
        scratch_types=[
            pltpu.VMEM((rows_per_chunk, w), jnp.float32),
            pltpu.VMEM((rows_per_chunk, w), jnp.float32),
            pltpu.VMEM((rows_per_chunk, w), jnp.float32),
            pltpu.VMEM((rows_per_chunk, w), jnp.float32),
            pltpu.VMEM((n_channels * _TAB,), jnp.int32),
            pltpu.VMEM((_pad128(n_channels),), jnp.float32),
            pltpu.VMEM((_pad128(n_channels),), jnp.float32),
            pltpu.SemaphoreType.DMA,
            pltpu.SemaphoreType.DMA,
            pltpu.SemaphoreType.DMA,
            pltpu.SemaphoreType.DMA,
        ],
        compiler_params=pltpu.CompilerParams(
            needs_layout_passes=False, use_tc_tiling_on_sc=True),
    )
    def pwlu(x_hbm, ab_hbm, lo_hbm, inv_hbm, out_hbm,
             xb0, xb1, yb0, yb1, abv, lov, invv, in0, in1, out0, out1):
        wid = lax.axis_index("s") * n_cores + lax.axis_index("c")
        base_chunk = wid * chunks_per_tile

        pltpu.sync_copy(ab_hbm, abv)
        pltpu.sync_copy(lo_hbm, lov)
        pltpu.sync_copy(inv_hbm, invv)

        xbufs = (xb0, xb1)
        ybufs = (yb0, yb1)
        isems = (in0, in1)
        osems = (out0, out1)

        def addr(k):
            q = k % chunks_per_plane
            plane = k // chunks_per_plane
            c = plane % n_channels
            b = plane // n_channels
            return b, c, q * rows_per_chunk

        def in_copy(bf, k):
            b, c, h0 = addr(k)
            return pltpu.make_async_copy(
                x_hbm.at[b, c, pl.ds(h0, rows_per_chunk), :], xbufs[bf], isems[bf])

        def out_copy(bf, k):
            b, c, h0 = addr(k)
            return pltpu.make_async_copy(
                ybufs[bf], out_hbm.at[b, c, pl.ds(h0, rows_per_chunk), :], osems[bf])

        def compute(bf, k):
            c = (k // chunks_per_plane) % n_channels
            ci = jnp.full((_LANES,), c, jnp.int32)
            lo = plsc.load_gather(lov, [ci])
            inv = plsc.load_gather(invv, [ci])
            cbase = ci * _TAB
            xb = xbufs[bf]
            yb = ybufs[bf]

            @plsc.parallel_loop(0, rows_per_chunk, step=1, unroll=2)
            def _(r):
                for o in range(w // _LANES):
                    off = o * _LANES
                    xv = xb[r, pl.ds(off, _LANES)]
                    xn = (xv - lo) * inv
                    xc = jnp.minimum(jnp.maximum(xn, 0.0), 1.001)
                    ri = (xc * np1).astype(jnp.int32) + cbase
                    abp = plsc.load_gather(abv, [ri])
                    avv = plsc.bitcast(
                        abp & jnp.full((_LANES,), -65536, jnp.int32), jnp.float32)
                    bvv = plsc.bitcast(abp << 16, jnp.float32)
                    yb[r, pl.ds(off, _LANES)] = avv + xv * bvv

        in_copy(0, base_chunk).start()
        in_copy(1, base_chunk + 1).start()

        @pl.loop(0, n_pairs)
        def _(p):
            for bf in (0, 1):
                k = base_chunk + 2 * p + bf
                in_copy(bf, k).wait()

                @pl.when(p >= 1)
                def _():
                    out_copy(bf, k - 2).wait()

                compute(bf, k)
                out_copy(bf, k).start()

                @pl.when(p < n_pairs - 1)
                def _():
                    in_copy(bf, k + 2).start()

        out_copy(0, base_chunk + chunks_per_tile - 2).wait()
        out_copy(1, base_chunk + chunks_per_tile - 1).wait()

    return pwlu(x, ab_tab, lo_tab, inv_tab)


def kernel(x, points, bounds, left_slopes, right_slopes):
    bsz, n_channels, h, w = x.shape
    n_points = points.shape[-1]
    n_regions = n_points - 1
    f32 = jnp.float32

    left_b = bounds[:, 0].astype(f32)
    right_b = bounds[:, 1].astype(f32)
    region_len = (right_b - left_b) / n_regions
    false_points = jnp.concatenate(
        [(points[:, 0] - left_slopes * region_len)[:, None], points], axis=1)
    inner = (points[:, 1:] - points[:, :-1]) / region_len[:, None]
    slopes = jnp.concatenate(
        [left_slopes[:, None], inner, right_slopes[:, None]], axis=1)
    sim_left = left_b - region_len
    ridx = jnp.arange(n_points + 1, dtype=f32)
    a_tab = false_points - (sim_left[:, None] + ridx[None, :] * region_len[:, None]) * slopes
    a16 = jax.lax.bitcast_convert_type(a_tab.astype(jnp.bfloat16), jnp.uint16).astype(jnp.uint32)
    b16 = jax.lax.bitcast_convert_type(slopes.astype(jnp.bfloat16), jnp.uint16).astype(jnp.uint32)
    ab_tab = jax.lax.bitcast_convert_type((a16 << 16) | b16, jnp.int32)
    ab_pad = jnp.zeros((n_channels, _TAB), jnp.int32).at[:, : n_points + 1].set(ab_tab).reshape(-1)
    inv_norm = 1.0 / ((n_regions + 1) * region_len)
    cpad = _pad128(n_channels)
    lo_pad = jnp.zeros((cpad,), f32).at[:n_channels].set(sim_left)
    inv_pad = jnp.zeros((cpad,), f32).at[:n_channels].set(inv_norm)

    info = plsc.get_sparse_core_info()
    nw = info.num_cores * info.num_subcores

    return _pwlu_sc(
        x, ab_pad, lo_pad, inv_pad,
        rows_per_chunk=56, n_channels=n_channels, nw=nw, n_cores=info.num_cores,
        np1=float(n_regions + 1))


# speedup vs baseline: 1.0871x; 1.0871x over previous
import functools

import jax
import jax.numpy as jnp
from jax import lax
from jax.experimental import pallas as pl
from jax.experimental.pallas import tpu as pltpu
from jax.experimental.pallas import tpu_sc as plsc

_LANES = 16
_TAB = 32
_NBUF = 3


def _pad128(n):
    return (n + 127) // 128 * 128


@functools.partial(
    jax.jit,
    static_argnames=("rows_per_chunk", "n_channels", "nw", "n_cores", "rmax"),
)
def _pwlu_sc(x, a_tab, b_tab, lo_tab, inv_tab, *, rows_per_chunk,
             n_channels, nw, n_cores, rmax):
    bsz, _, h, w = x.shape
    chunks_per_plane = h // rows_per_chunk
    n_chunks = bsz * n_channels * chunks_per_plane
    chunks_per_tile = n_chunks // nw
    n_trips = chunks_per_tile // _NBUF
    mesh = plsc.VectorSubcoreMesh(core_axis_name="c", subcore_axis_name="s")

    @functools.partial(
        pl.kernel,
        out_type=jax.ShapeDtypeStruct(x.shape, jnp.float32),
        mesh=mesh,
        scratch_types=[
            pltpu.VMEM((_NBUF, rows_per_chunk, w), jnp.float32),
            pltpu.VMEM((_NBUF, rows_per_chunk, w), jnp.float32),
            pltpu.VMEM((n_channels * _TAB,), jnp.float32),
            pltpu.VMEM((n_channels * _TAB,), jnp.float32),
            pltpu.VMEM((_pad128(n_channels),), jnp.float32),
            pltpu.VMEM((_pad128(n_channels),), jnp.float32),
            [pltpu.SemaphoreType.DMA] * _NBUF,
            [pltpu.SemaphoreType.DMA] * _NBUF,
        ],
        compiler_params=pltpu.CompilerParams(
            needs_layout_passes=False, use_tc_tiling_on_sc=True),
    )
    def pwlu(x_hbm, a_hbm, b_hbm, lo_hbm, inv_hbm, out_hbm,
             xb, yb, av, bv, lov, invv, isems, osems):
        wid = lax.axis_index("s") * n_cores + lax.axis_index("c")
        base_chunk = wid * chunks_per_tile

        pltpu.sync_copy(a_hbm, av)
        pltpu.sync_copy(b_hbm, bv)
        pltpu.sync_copy(lo_hbm, lov)
        pltpu.sync_copy(inv_hbm, invv)

        def addr(k):
            q = k % chunks_per_plane
            plane = k // chunks_per_plane
            c = plane % n_channels
            b = plane // n_channels
            return b, c, q * rows_per_chunk

        def in_copy(bf, k):
            b, c, h0 = addr(k)
            return pltpu.make_async_copy(
                x_hbm.at[b, c, pl.ds(h0, rows_per_chunk), :], xb.at[bf], isems[bf])

        def out_copy(bf, k):
            b, c, h0 = addr(k)
            return pltpu.make_async_copy(
                yb.at[bf], out_hbm.at[b, c, pl.ds(h0, rows_per_chunk), :], osems[bf])

        def compute(bf, k):
            c = (k // chunks_per_plane) % n_channels
            ci = jnp.full((_LANES,), c, jnp.int32)
            lo = plsc.load_gather(lov, [ci])
            inv = plsc.load_gather(invv, [ci])
            cbase = ci * _TAB

            @plsc.parallel_loop(0, rows_per_chunk, step=1, unroll=2)
            def _(r):
                for o in range(w // _LANES):
                    off = o * _LANES
                    xv = xb[bf, r, pl.ds(off, _LANES)]
                    xn = (xv - lo) * inv
                    xc = jnp.minimum(jnp.maximum(xn, 0.0), rmax)
                    ri = xc.astype(jnp.int32) + cbase
                    avv = plsc.load_gather(av, [ri])
                    bvv = plsc.load_gather(bv, [ri])
                    yb[bf, r, pl.ds(off, _LANES)] = avv + xv * bvv

        for bf in range(_NBUF):
            in_copy(bf, base_chunk + bf).start()

        @pl.loop(0, n_trips)
        def _(p):
            for bf in range(_NBUF):
                k = base_chunk + _NBUF * p + bf
                in_copy(bf, k).wait()

                @pl.when(p >= 1)
                def _():
                    out_copy(bf, k - _NBUF).wait()

                compute(bf, k)
                out_copy(bf, k).start()

                @pl.when(p < n_trips - 1)
                def _():
                    in_copy(bf, k + _NBUF).start()

        for bf in range(_NBUF):
            out_copy(bf, base_chunk + chunks_per_tile - _NBUF + bf).wait()

    return pwlu(x, a_tab, b_tab, lo_tab, inv_tab)


def kernel(x, points, bounds, left_slopes, right_slopes):
    bsz, n_channels, h, w = x.shape
    n_points = points.shape[-1]
    n_regions = n_points - 1
    f32 = jnp.float32

    left_b = bounds[:, 0].astype(f32)
    right_b = bounds[:, 1].astype(f32)
    region_len = (right_b - left_b) / n_regions
    false_points = jnp.concatenate(
        [(points[:, 0] - left_slopes * region_len)[:, None], points], axis=1)
    inner = (points[:, 1:] - points[:, :-1]) / region_len[:, None]
    slopes = jnp.concatenate(
        [left_slopes[:, None], inner, right_slopes[:, None]], axis=1)
    sim_left = left_b - region_len
    ridx = jnp.arange(n_points + 1, dtype=f32)
    a_tab = false_points - (sim_left[:, None] + ridx[None, :] * region_len[:, None]) * slopes
    a_pad = jnp.zeros((n_channels, _TAB), f32).at[:, : n_points + 1].set(a_tab).reshape(-1)
    b_pad = jnp.zeros((n_channels, _TAB), f32).at[:, : n_points + 1].set(slopes).reshape(-1)
    inv_norm = 1.0 / region_len
    cpad = _pad128(n_channels)
    lo_pad = jnp.zeros((cpad,), f32).at[:n_channels].set(sim_left)
    inv_pad = jnp.zeros((cpad,), f32).at[:n_channels].set(inv_norm)

    info = plsc.get_sparse_core_info()
    nw = info.num_cores * info.num_subcores

    return _pwlu_sc(
        x, a_pad, b_pad, lo_pad, inv_pad,
        rows_per_chunk=56, n_channels=n_channels, nw=nw, n_cores=info.num_cores,
        rmax=float(n_regions + 1) + 0.01)


# speedup vs baseline: 1.2149x; 1.1176x over previous
import functools

import jax
import jax.numpy as jnp
from jax import lax
from jax.experimental import pallas as pl
from jax.experimental.pallas import tpu as pltpu
from jax.experimental.pallas import tpu_sc as plsc

_LANES = 16
_TAB = 32
_NBUF = 2


def _pad128(n):
    return (n + 127) // 128 * 128


@functools.partial(
    jax.jit,
    static_argnames=("rows_per_chunk", "n_channels", "nw", "n_cores", "rmax"),
)
def _pwlu_sc(x, a_tab, b_tab, lo_tab, inv_tab, *, rows_per_chunk,
             n_channels, nw, n_cores, rmax):
    bsz, _, h, w = x.shape
    chunks_per_plane = h // rows_per_chunk
    n_chunks = bsz * n_channels * chunks_per_plane
    chunks_per_tile = n_chunks // nw
    n_trips = chunks_per_tile // _NBUF
    mesh = plsc.VectorSubcoreMesh(core_axis_name="c", subcore_axis_name="s")

    @functools.partial(
        pl.kernel,
        out_type=jax.ShapeDtypeStruct(x.shape, jnp.float32),
        mesh=mesh,
        scratch_types=[
            pltpu.VMEM((_NBUF, rows_per_chunk, w), jnp.float32),
            pltpu.VMEM((_NBUF, rows_per_chunk, w), jnp.float32),
            pltpu.VMEM((n_channels * _TAB,), jnp.float32),
            pltpu.VMEM((n_channels * _TAB,), jnp.float32),
            pltpu.VMEM((_pad128(n_channels),), jnp.float32),
            pltpu.VMEM((_pad128(n_channels),), jnp.float32),
            [pltpu.SemaphoreType.DMA] * _NBUF,
            [pltpu.SemaphoreType.DMA] * _NBUF,
            pltpu.SemaphoreType.DMA,
        ],
        compiler_params=pltpu.CompilerParams(
            needs_layout_passes=False, use_tc_tiling_on_sc=True),
    )
    def pwlu(x_hbm, a_hbm, b_hbm, lo_hbm, inv_hbm, out_hbm,
             xb, yb, av, bv, lov, invv, isems, osems, tsem):
        wid = lax.axis_index("s") * n_cores + lax.axis_index("c")
        base_chunk = wid * chunks_per_tile

        tab_copies = (
            pltpu.make_async_copy(a_hbm, av, tsem),
            pltpu.make_async_copy(b_hbm, bv, tsem),
            pltpu.make_async_copy(lo_hbm, lov, tsem),
            pltpu.make_async_copy(inv_hbm, invv, tsem),
        )
        for cp in tab_copies:
            cp.start()

        def addr(k):
            q = k % chunks_per_plane
            plane = k // chunks_per_plane
            c = plane % n_channels
            b = plane // n_channels
            return b, c, q * rows_per_chunk

        def in_copy(bf, k):
            b, c, h0 = addr(k)
            return pltpu.make_async_copy(
                x_hbm.at[b, c, pl.ds(h0, rows_per_chunk), :], xb.at[bf], isems[bf])

        def out_copy(bf, k):
            b, c, h0 = addr(k)
            return pltpu.make_async_copy(
                yb.at[bf], out_hbm.at[b, c, pl.ds(h0, rows_per_chunk), :], osems[bf])

        def compute(bf, k):
            c = (k // chunks_per_plane) % n_channels
            ci = jnp.full((_LANES,), c, jnp.int32)
            lo = plsc.load_gather(lov, [ci])
            inv = plsc.load_gather(invv, [ci])
            cbase = ci * _TAB

            @plsc.parallel_loop(0, rows_per_chunk, step=1, unroll=2)
            def _(r):
                for o in range(w // _LANES):
                    off = o * _LANES
                    xv = xb[bf, r, pl.ds(off, _LANES)]
                    xn = (xv - lo) * inv
                    xc = jnp.minimum(jnp.maximum(xn, 0.0), rmax)
                    ri = xc.astype(jnp.int32) + cbase
                    avv = plsc.load_gather(av, [ri])
                    bvv = plsc.load_gather(bv, [ri])
                    yb[bf, r, pl.ds(off, _LANES)] = avv + xv * bvv

        for bf in range(_NBUF):
            in_copy(bf, base_chunk + bf).start()
        for cp in tab_copies:
            cp.wait()

        @pl.loop(0, n_trips)
        def _(p):
            for bf in range(_NBUF):
                k = base_chunk + _NBUF * p + bf
                in_copy(bf, k).wait()

                @pl.when(p >= 1)
                def _():
                    out_copy(bf, k - _NBUF).wait()

                compute(bf, k)
                out_copy(bf, k).start()

                @pl.when(p < n_trips - 1)
                def _():
                    in_copy(bf, k + _NBUF).start()

        for bf in range(_NBUF):
            out_copy(bf, base_chunk + chunks_per_tile - _NBUF + bf).wait()

    return pwlu(x, a_tab, b_tab, lo_tab, inv_tab)


def kernel(x, points, bounds, left_slopes, right_slopes):
    bsz, n_channels, h, w = x.shape
    n_points = points.shape[-1]
    n_regions = n_points - 1
    f32 = jnp.float32

    left_b = bounds[:, 0].astype(f32)
    right_b = bounds[:, 1].astype(f32)
    region_len = (right_b - left_b) / n_regions
    false_points = jnp.concatenate(
        [(points[:, 0] - left_slopes * region_len)[:, None], points], axis=1)
    inner = (points[:, 1:] - points[:, :-1]) / region_len[:, None]
    slopes = jnp.concatenate(
        [left_slopes[:, None], inner, right_slopes[:, None]], axis=1)
    sim_left = left_b - region_len
    ridx = jnp.arange(n_points + 1, dtype=f32)
    a_tab = false_points - (sim_left[:, None] + ridx[None, :] * region_len[:, None]) * slopes
    a_pad = jnp.zeros((n_channels, _TAB), f32).at[:, : n_points + 1].set(a_tab).reshape(-1)
    b_pad = jnp.zeros((n_channels, _TAB), f32).at[:, : n_points + 1].set(slopes).reshape(-1)
    inv_norm = 1.0 / region_len
    cpad = _pad128(n_channels)
    lo_pad = jnp.zeros((cpad,), f32).at[:n_channels].set(sim_left)
    inv_pad = jnp.zeros((cpad,), f32).at[:n_channels].set(inv_norm)

    info = plsc.get_sparse_core_info()
    nw = info.num_cores * info.num_subcores

    return _pwlu_sc(
        x, a_pad, b_pad, lo_pad, inv_pad,
        rows_per_chunk=56, n_channels=n_channels, nw=nw, n_cores=info.num_cores,
        rmax=float(n_regions + 1) + 0.01)
